# baseline (speedup 1.0000x reference)
import jax
import jax.numpy as jnp
from jax.experimental import pallas as pl
from jax.experimental.pallas import tpu as pltpu

BM = 512
NBUF = 4


def _router_kernel(x_hbm, w_ref, b_ref, probs_ref, logits_ref, z_ref,
                   buf, sems):
    i = pl.program_id(0)
    n = pl.num_programs(0)

    def copy(block, slot):
        return pltpu.make_async_copy(
            x_hbm.at[pl.ds(block * BM, BM), :],
            buf.at[slot],
            sems.at[slot],
        )

    @pl.when(i == 0)
    def _prologue():
        for j in range(NBUF - 1):
            copy(j, j).start()

    nxt = i + NBUF - 1

    @pl.when(nxt < n)
    def _prefetch():
        copy(nxt, nxt % NBUF).start()

    slot = i % NBUF
    copy(i, slot).wait()

    logits = jax.lax.dot_general(
        buf[slot], w_ref[...],
        dimension_numbers=(((1,), (0,)), ((), ())),
        preferred_element_type=jnp.float32,
    )
    logits = logits + b_ref[...]
    logits_ref[...] = logits
    m = jnp.max(logits, axis=-1, keepdims=True)
    e = jnp.exp(logits - m)
    s = jnp.sum(e, axis=-1, keepdims=True)
    probs_ref[...] = e / s
    log_z = jnp.log(s) + m
    z_ref[...] = jnp.sum(log_z * log_z).reshape(1, 1, 1)


def kernel(token_inputs, W, b, num_experts, expert_capacity):
    G, T, H = token_inputs.shape
    E = W.shape[1]
    M = G * T
    x = token_inputs.reshape(M, H)
    N = M // BM

    probs, logits, zparts = pl.pallas_call(
        _router_kernel,
        grid=(N,),
        in_specs=[
            pl.BlockSpec(memory_space=pl.ANY),
            pl.BlockSpec((H, E), lambda i: (0, 0)),
            pl.BlockSpec((1, E), lambda i: (0, 0)),
        ],
        out_specs=[
            pl.BlockSpec((BM, E), lambda i: (i, 0)),
            pl.BlockSpec((BM, E), lambda i: (i, 0)),
            pl.BlockSpec((1, 1, 1), lambda i: (i, 0, 0)),
        ],
        out_shape=[
            jax.ShapeDtypeStruct((M, E), jnp.float32),
            jax.ShapeDtypeStruct((M, E), jnp.float32),
            jax.ShapeDtypeStruct((N, 1, 1), jnp.float32),
        ],
        scratch_shapes=[
            pltpu.VMEM((NBUF, BM, H), jnp.float32),
            pltpu.SemaphoreType.DMA((NBUF,)),
        ],
    )(x, W, b.reshape(1, E))

    z_loss = jnp.sum(zparts) / M
    return probs.reshape(G, T, E), logits.reshape(G, T, E), z_loss


# speedup vs baseline: 1.0306x; 1.0306x over previous
import jax
import jax.numpy as jnp
from jax.experimental import pallas as pl


def _router_kernel(x_ref, w_ref, b_ref, probs_ref, logits_ref, z_ref):
    xb = x_ref[...].astype(jnp.bfloat16)
    wb = w_ref[...].astype(jnp.bfloat16)
    logits = jax.lax.dot_general(
        xb, wb,
        dimension_numbers=(((1,), (0,)), ((), ())),
        preferred_element_type=jnp.float32,
    )
    logits = logits + b_ref[...]
    logits_ref[...] = logits
    m = jnp.max(logits, axis=-1, keepdims=True)
    e = jnp.exp(logits - m)
    s = jnp.sum(e, axis=-1, keepdims=True)
    probs_ref[...] = e / s
    log_z = jnp.log(s) + m
    z_ref[...] = jnp.sum(log_z * log_z).reshape(1, 1, 1)


def kernel(token_inputs, W, b, num_experts, expert_capacity):
    G, T, H = token_inputs.shape
    E = W.shape[1]
    M = G * T
    x = token_inputs.reshape(M, H)
    BM = 1024
    N = M // BM

    probs, logits, zparts = pl.pallas_call(
        _router_kernel,
        grid=(N,),
        in_specs=[
            pl.BlockSpec((BM, H), lambda i: (i, 0)),
            pl.BlockSpec((H, E), lambda i: (0, 0)),
            pl.BlockSpec((1, E), lambda i: (0, 0)),
        ],
        out_specs=[
            pl.BlockSpec((BM, E), lambda i: (i, 0)),
            pl.BlockSpec((BM, E), lambda i: (i, 0)),
            pl.BlockSpec((1, 1, 1), lambda i: (i, 0, 0)),
        ],
        out_shape=[
            jax.ShapeDtypeStruct((M, E), jnp.float32),
            jax.ShapeDtypeStruct((M, E), jnp.float32),
            jax.ShapeDtypeStruct((N, 1, 1), jnp.float32),
        ],
    )(x, W, b.reshape(1, E))

    z_loss = jnp.sum(zparts) / M
    return probs.reshape(G, T, E), logits.reshape(G, T, E), z_loss
